# Initial kernel scaffold; baseline (speedup 1.0000x reference)
import jax
import jax.numpy as jnp
from jax.experimental import pallas as pl


def kernel(emb_feat_0, emb_feat_1, emb_feat_2, num_feat_0, num_feat_1, event_time, seq_lens, emb_table_0, emb_table_1, emb_table_2, bn_gamma_0, bn_beta_0, bn_gamma_1, bn_beta_1, lin_w_0, lin_b_0, lin_w_1, lin_b_1):
    raise NotImplementedError("write your pallas kernel here")


# speedup vs baseline: 1.8859x; 1.8859x over previous
import functools

import jax
import jax.numpy as jnp
from jax import lax
from jax.experimental import pallas as pl
from jax.experimental.pallas import tpu as pltpu
from jax.experimental.pallas import tpu_sc as plsc

_B, _T = 1024, 200
_EMB = 64
_NE = 16
_EPS = 1e-5

_NC, _NS = 2, 16
_NW = _NC * _NS
_ROWS = _B * _T
_RPW = _ROWS // _NW
_CH = 800
_NCHUNK = _RPW // _CH
_OUTC = 3 * _EMB + 2 * _NE


def _stats_body(x0_ref, x1_ref, sl_ref, gb_ref, out_ref):
    sl = sl_ref[...]
    t = lax.broadcasted_iota(jnp.int32, (_B, _T), 1)
    m = (t < sl).astype(jnp.float32)
    cnt = jnp.maximum(jnp.sum(m), 1.0)
    for i, x_ref in enumerate((x0_ref, x1_ref)):
        x = x_ref[...]
        mean = jnp.sum(x * m) / cnt
        var = jnp.sum(((x - mean) ** 2) * m) / cnt
        gamma = gb_ref[0, 2 * i]
        beta = gb_ref[0, 2 * i + 1]
        scale = gamma * lax.rsqrt(var + _EPS)
        out_ref[0, 2 * i] = scale
        out_ref[0, 2 * i + 1] = beta - mean * scale


def _stats_call(x0, x1, sl2, gb):
    return pl.pallas_call(
        _stats_body,
        out_shape=jax.ShapeDtypeStruct((1, 4), jnp.float32),
        in_specs=[
            pl.BlockSpec((_B, _T), lambda: (0, 0)),
            pl.BlockSpec((_B, _T), lambda: (0, 0)),
            pl.BlockSpec((_B, 1), lambda: (0, 0)),
            pl.BlockSpec(memory_space=pltpu.SMEM),
        ],
        out_specs=pl.BlockSpec(memory_space=pltpu.SMEM),
    )(x0, x1, sl2, gb)


_PAIR = 2 * _EMB


def _sc_body(i0, i1, i2, t0, t1, t2, o0, o1, o2, idx_v, rows_v, sem):
    wid = lax.axis_index("s") * _NC + lax.axis_index("c")
    base = wid * _RPW

    def chunk_body(c, carry):
        off = base + c * _CH
        for iref, tref, oref in ((i0, t0, o0), (i1, t1, o1), (i2, t2, o2)):
            pltpu.sync_copy(iref.at[pl.ds(off, _CH)], idx_v)
            pltpu.async_copy(tref.at[idx_v], rows_v, sem).wait()
            pltpu.sync_copy(rows_v, oref.at[pl.ds(off, _CH)])
        return carry

    lax.fori_loop(0, _NCHUNK, chunk_body, 0)


_sc_gather = functools.partial(
    pl.kernel,
    _sc_body,
    out_type=[jax.ShapeDtypeStruct((_ROWS, _PAIR), jnp.float32)] * 3,
    mesh=plsc.VectorSubcoreMesh(
        core_axis_name="c", subcore_axis_name="s",
        num_cores=_NC, num_subcores=_NS),
    scratch_types=[
        pltpu.VMEM((_CH,), jnp.int32),
        pltpu.VMEM((_CH, _PAIR), jnp.float32),
        pltpu.SemaphoreType.DMA,
    ],
)()


_RB = 8 * _T


def _fuse_body(c0_ref, c1_ref, c2_ref, e0_ref, e1_ref, e2_ref,
               x0_ref, x1_ref, sl_ref, wb_ref, sc_ref, out_ref):
    t = lax.broadcasted_iota(jnp.int32, (_RB, 1), 0) % _T
    mask = t < sl_ref[...]
    wb = wb_ref[...]
    cats = []
    for c_ref, e_ref in ((c0_ref, e0_ref), (c1_ref, e1_ref), (c2_ref, e2_ref)):
        c = c_ref[...]
        odd = (e_ref[...] & 1) == 1
        cats.append(jnp.where(odd, c[:, _EMB:], c[:, :_EMB]))
    halves = []
    for i, x_ref in enumerate((x0_ref, x1_ref)):
        x = x_ref[...]
        n = jnp.where(mask, x * sc_ref[0, 2 * i] + sc_ref[0, 2 * i + 1], x)
        w = wb[:, 32 * i:32 * i + _NE]
        b = wb[:, 32 * i + _NE:32 * i + 32]
        halves.append(n * w + b)
    out_ref[...] = jnp.concatenate(cats + halves, axis=-1)


def _fuse_call(c0, c1, c2, e0, e1, e2, x0, x1, slr, wb, sc):
    cat_spec = pl.BlockSpec((_RB, _PAIR), lambda i: (i, 0))
    col_spec = pl.BlockSpec((_RB, 1), lambda i: (i, 0))
    return pl.pallas_call(
        _fuse_body,
        grid=(_ROWS // _RB,),
        out_shape=jax.ShapeDtypeStruct((_ROWS, _OUTC), jnp.float32),
        in_specs=[
            cat_spec, cat_spec, cat_spec,
            col_spec, col_spec, col_spec,
            col_spec, col_spec,
            col_spec,
            pl.BlockSpec((1, 64), lambda i: (0, 0)),
            pl.BlockSpec(memory_space=pltpu.SMEM),
        ],
        out_specs=pl.BlockSpec((_RB, _OUTC), lambda i: (i, 0)),
    )(c0, c1, c2, e0, e1, e2, x0, x1, slr, wb, sc)


def kernel(emb_feat_0, emb_feat_1, emb_feat_2, num_feat_0, num_feat_1,
           event_time, seq_lens, emb_table_0, emb_table_1, emb_table_2,
           bn_gamma_0, bn_beta_0, bn_gamma_1, bn_beta_1,
           lin_w_0, lin_b_0, lin_w_1, lin_b_1):
    i0 = emb_feat_0.astype(jnp.int32).reshape(_ROWS)
    i1 = emb_feat_1.astype(jnp.int32).reshape(_ROWS)
    i2 = emb_feat_2.astype(jnp.int32).reshape(_ROWS)
    x0 = num_feat_0.astype(jnp.float32)
    x1 = num_feat_1.astype(jnp.float32)
    sl2 = seq_lens.astype(jnp.int32).reshape(_B, 1)
    gb = jnp.stack([bn_gamma_0.astype(jnp.float32).reshape(()),
                    bn_beta_0.astype(jnp.float32).reshape(()),
                    bn_gamma_1.astype(jnp.float32).reshape(()),
                    bn_beta_1.astype(jnp.float32).reshape(())]).reshape(1, 4)
    wb = jnp.concatenate([lin_w_0.astype(jnp.float32).reshape(_NE),
                          lin_b_0.astype(jnp.float32).reshape(_NE),
                          lin_w_1.astype(jnp.float32).reshape(_NE),
                          lin_b_1.astype(jnp.float32).reshape(_NE)]).reshape(1, 64)

    sc = _stats_call(x0, x1, sl2, gb)
    c0, c1, c2 = _sc_gather(
        i0 >> 1, i1 >> 1, i2 >> 1,
        emb_table_0.astype(jnp.float32).reshape(-1, _PAIR),
        emb_table_1.astype(jnp.float32).reshape(-1, _PAIR),
        emb_table_2.astype(jnp.float32).reshape(-1, _PAIR))
    slr = jnp.repeat(sl2.reshape(_B), _T).reshape(_ROWS, 1)
    out = _fuse_call(c0, c1, c2,
                     i0.reshape(_ROWS, 1), i1.reshape(_ROWS, 1),
                     i2.reshape(_ROWS, 1),
                     x0.reshape(_ROWS, 1), x1.reshape(_ROWS, 1),
                     slr, wb, sc)
    return (out.reshape(_B, _T, _OUTC), event_time.astype(jnp.float32))
